# baseline (speedup 1.0000x reference)
import functools

import jax
import jax.numpy as jnp
from jax import lax
from jax.experimental import pallas as pl
from jax.experimental.pallas import tpu as pltpu
from jax.experimental.pallas import tpu_sc as plsc

NSUB = 16
NCORE = 2
LANES = 16


def _splat(vec16, j):
    return lax.gather(
        vec16, jnp.full((LANES, 1), j, jnp.int32),
        lax.GatherDimensionNumbers(offset_dims=(), collapsed_slice_dims=(0,),
                                   start_index_map=(0,)),
        (1,), mode=lax.GatherScatterMode.PROMISE_IN_BOUNDS)


CH = 128
NBUF = 6


def _chunks_per_tile(e):
    return -(-e // (NSUB * CH * 2 * NBUF)) * 2 * NBUF


@functools.lru_cache(maxsize=None)
def _make_spmm(n, e, hd):
    assert n % NSUB == 0 and hd % LANES == 0
    cpt = _chunks_per_tile(e)
    nsteps = cpt // (2 * NBUF)
    rpa = -(-(n // NSUB) // 8) * 8
    rpl = n - (NSUB - 1) * rpa
    assert 0 < rpl <= rpa and rpl % 8 == 0
    nslice = hd // LANES

    mesh = plsc.VectorSubcoreMesh(core_axis_name="c", subcore_axis_name="s",
                                  num_cores=NCORE, num_subcores=NSUB)

    @functools.partial(
        pl.kernel,
        out_type=(jax.ShapeDtypeStruct((n, hd), jnp.float32),
                  jax.ShapeDtypeStruct((n, hd), jnp.float32)),
        mesh=mesh,
        scratch_types=[
            pltpu.VMEM_SHARED((n, hd), jnp.float32),
            pltpu.VMEM((NBUF, 3, CH), jnp.int32),
            pltpu.VMEM((NBUF, 3, CH), jnp.int32),
            [pltpu.VMEM((CH, hd), jnp.float32) for _ in range(NBUF)],
            [pltpu.SemaphoreType.DMA for _ in range(NBUF)],
            [pltpu.SemaphoreType.DMA for _ in range(NBUF)],
            pltpu.SemaphoreType.DMA,
            pltpu.SemaphoreType.DMA,
        ],
        compiler_params=pltpu.CompilerParams(use_tc_tiling_on_sc=False,
                                             needs_layout_passes=False),
    )
    def spmm(yl, yr, packed_h, zero_h, zl, zr,
             acc, ib0, ib1, msgs, gsems, ssems, isem0, isem1):
        sub = lax.axis_index("s")
        core = lax.axis_index("c")
        cbase = sub * cpt

        @pl.when(sub < NSUB - 1)
        def _():
            pltpu.sync_copy(zero_h, acc.at[pl.ds(sub * rpa, rpa)])

        @pl.when(sub == NSUB - 1)
        def _():
            pltpu.sync_copy(zero_h.at[pl.ds(0, rpl)],
                            acc.at[pl.ds(sub * rpa, rpl)])
        plsc.subcore_barrier()

        def scale(msg, ib, b):
            def group(g, carry):
                w16 = plsc.bitcast(ib[b, 2, pl.ds(g * LANES, LANES)],
                                   jnp.float32)
                for j in range(LANES):
                    k = g * LANES + j
                    s = _splat(w16, j)
                    for c in range(nslice):
                        sl = pl.ds(c * LANES, LANES)
                        msg[k, sl] = msg[k, sl] * s
                return carry
            lax.fori_loop(0, CH // LANES, group, 0)

        for half in range(NCORE):
            @pl.when(core == half)
            def _():
                ytab = (yl, yr)[half]

                def drain_scatter(b):
                    pltpu.make_async_copy(zero_h.at[pl.ds(0, CH)], msgs[b],
                                          ssems[b]).wait()

                def halfstep(ib, ib_other, isem_other, base_other):
                    descs = []
                    for b in range(NBUF):
                        drain_scatter(b)
                        descs.append(pltpu.async_copy(
                            ytab.at[ib.at[b, 1]], msgs[b], gsems[b]))
                    pref = pltpu.async_copy(
                        packed_h.at[pl.ds(base_other, NBUF)],
                        ib_other, isem_other)
                    for b in range(NBUF):
                        descs[b].wait()
                        scale(msgs[b], ib, b)
                        pltpu.async_copy(msgs[b], acc.at[ib.at[b, 0]],
                                         ssems[b], add=True)
                    pref.wait()

                def step(j, carry):
                    base = cbase + 2 * NBUF * j
                    halfstep(ib0, ib1, isem1, base + NBUF)
                    halfstep(ib1, ib0, isem0, base + 2 * NBUF)
                    return carry

                for b in range(NBUF):
                    pltpu.async_copy(zero_h.at[pl.ds(0, CH)], msgs[b],
                                     ssems[b])
                pltpu.sync_copy(packed_h.at[pl.ds(cbase, NBUF)], ib0)
                lax.fori_loop(0, nsteps, step, 0)
                for b in range(NBUF):
                    drain_scatter(b)

        plsc.subcore_barrier()
        for half, zout in enumerate((zl, zr)):
            @pl.when((core == half) & (sub < NSUB - 1))
            def _():
                pltpu.sync_copy(acc.at[pl.ds(sub * rpa, rpa)],
                                zout.at[pl.ds(sub * rpa, rpa)])

            @pl.when((core == half) & (sub == NSUB - 1))
            def _():
                pltpu.sync_copy(acc.at[pl.ds(sub * rpa, rpl)],
                                zout.at[pl.ds(sub * rpa, rpl)])

    return spmm


def _dot(a, b):
    return jax.lax.dot_general(
        a, b, (((1,), (0,)), ((), ())),
        precision=jax.lax.Precision.HIGHEST,
        preferred_element_type=jnp.float32)


def _pack_half(y4, lo, hd):
    return jnp.concatenate([y4[:, i, lo:lo + hd] for i in range(4)], axis=1)


def _unpack_z(zl, zr, bm, hd):
    quarters = [
        jnp.concatenate([zl[:, i * hd:(i + 1) * hd],
                         zr[:, i * hd:(i + 1) * hd]], axis=1)
        for i in range(4)]
    return jnp.stack(quarters, axis=1).reshape(bm, 2 * hd)


def _pack_x(x, bm, d):
    xv = x.reshape(bm // 2, 2, d)
    return jnp.concatenate([xv[:, 0, :], xv[:, 1, :]], axis=1)


def _unpack_x(xp, bm, d):
    return jnp.stack([xp[:, :d], xp[:, d:]], axis=1).reshape(bm, d)


@functools.lru_cache(maxsize=None)
def _make_t_first(n, d, bm):
    hd = d // 2

    def body(x_ref, w_ref, yl_ref, yr_ref):
        y = _dot(x_ref[...], w_ref[...].T)
        y4 = y.reshape(bm // 4, 4, d)
        yl_ref[...] = _pack_half(y4, 0, hd)
        yr_ref[...] = _pack_half(y4, hd, hd)

    return pl.pallas_call(
        body,
        grid=(-(-n // bm),),
        in_specs=[pl.BlockSpec((bm, d), lambda i: (i, 0)),
                  pl.BlockSpec((d, d), lambda i: (0, 0))],
        out_specs=[pl.BlockSpec((bm // 4, 4 * hd), lambda i: (i, 0)),
                   pl.BlockSpec((bm // 4, 4 * hd), lambda i: (i, 0))],
        out_shape=[jax.ShapeDtypeStruct((n // 4, 4 * hd), jnp.float32),
                   jax.ShapeDtypeStruct((n // 4, 4 * hd), jnp.float32)],
        compiler_params=pltpu.CompilerParams(
            dimension_semantics=("parallel",)),
    )


@functools.lru_cache(maxsize=None)
def _make_t_mid(n, d, bm, residual):
    hd = d // 2

    def body(*refs):
        if residual:
            zl_ref, zr_ref, xp_ref, w_ref, x_ref, yl_ref, yr_ref = refs
        else:
            zl_ref, zr_ref, w_ref, x_ref, yl_ref, yr_ref = refs
        z = _unpack_z(zl_ref[...], zr_ref[...], bm, hd)
        x = jnp.maximum(z, 0.0)
        if residual:
            x = x + xp_ref[...]
        x_ref[...] = x
        y = _dot(x, w_ref[...].T)
        y4 = y.reshape(bm // 4, 4, d)
        yl_ref[...] = _pack_half(y4, 0, hd)
        yr_ref[...] = _pack_half(y4, hd, hd)

    in_specs = [pl.BlockSpec((bm // 4, 4 * hd), lambda i: (i, 0)),
                pl.BlockSpec((bm // 4, 4 * hd), lambda i: (i, 0))]
    if residual:
        in_specs.append(pl.BlockSpec((bm, d), lambda i: (i, 0)))
    in_specs.append(pl.BlockSpec((d, d), lambda i: (0, 0)))

    return pl.pallas_call(
        body,
        grid=(-(-n // bm),),
        in_specs=in_specs,
        out_specs=[pl.BlockSpec((bm, d), lambda i: (i, 0)),
                   pl.BlockSpec((bm // 4, 4 * hd), lambda i: (i, 0)),
                   pl.BlockSpec((bm // 4, 4 * hd), lambda i: (i, 0))],
        out_shape=[jax.ShapeDtypeStruct((n, d), jnp.float32),
                   jax.ShapeDtypeStruct((n // 4, 4 * hd), jnp.float32),
                   jax.ShapeDtypeStruct((n // 4, 4 * hd), jnp.float32)],
        compiler_params=pltpu.CompilerParams(
            dimension_semantics=("parallel",)),
    )


@functools.lru_cache(maxsize=None)
def _make_t_final(n, d, bm):
    hd = d // 2

    def body(zl_ref, zr_ref, x2_ref, x0_ref, x1_ref, wc_ref, bc_ref, o_ref):
        z = _unpack_z(zl_ref[...], zr_ref[...], bm, hd)
        x3 = jnp.maximum(z, 0.0) + x2_ref[...]
        comb = jnp.concatenate(
            [x0_ref[...], x1_ref[...], x2_ref[...], x3], axis=1)
        o_ref[...] = _dot(comb, wc_ref[...].T) + bc_ref[...]

    return pl.pallas_call(
        body,
        grid=(-(-n // bm),),
        in_specs=[pl.BlockSpec((bm // 4, 4 * hd), lambda i: (i, 0)),
                  pl.BlockSpec((bm // 4, 4 * hd), lambda i: (i, 0)),
                  pl.BlockSpec((bm, d), lambda i: (i, 0)),
                  pl.BlockSpec((bm, d), lambda i: (i, 0)),
                  pl.BlockSpec((bm, d), lambda i: (i, 0)),
                  pl.BlockSpec((d, 4 * d), lambda i: (0, 0)),
                  pl.BlockSpec((1, d), lambda i: (0, 0))],
        out_specs=pl.BlockSpec((bm, d), lambda i: (i, 0)),
        out_shape=jax.ShapeDtypeStruct((n, d), jnp.float32),
        compiler_params=pltpu.CompilerParams(
            dimension_semantics=("parallel",)),
    )


def kernel(base_emb, edge_index, edge_weight, W0, W1, W2, Wc, bc):
    n, d = base_emb.shape
    e = edge_weight.shape[0]
    hd = d // 2
    bm = 1024

    cpt = _chunks_per_tile(e)
    e2 = NSUB * cpt * CH
    pad = e2 - e
    rows = jnp.pad(edge_index[0], (0, pad))
    cols = jnp.pad(edge_index[1], (0, pad))
    wbits = jnp.pad(lax.bitcast_convert_type(edge_weight, jnp.int32),
                    (0, pad))
    packed = jnp.stack([rows.reshape(-1, CH), cols.reshape(-1, CH),
                        wbits.reshape(-1, CH)], axis=1)
    packed = jnp.pad(packed, ((0, 2 * NBUF), (0, 0), (0, 0)))
    zero = jnp.zeros((-(-(n // NSUB) // 8) * 8, hd), jnp.float32)
    bc2 = bc.reshape(1, d)

    spmm = _make_spmm(n, e, hd)
    t_first = _make_t_first(n, d, bm)
    t_mid_nores = _make_t_mid(n, d, bm, False)
    t_mid_res = _make_t_mid(n, d, bm, True)
    t_final = _make_t_final(n, d, bm)

    def sc(ylp, yrp):
        zl, zr = spmm(ylp.reshape(n, hd), yrp.reshape(n, hd), packed, zero)
        return zl.reshape(n // 4, 4 * hd), zr.reshape(n // 4, 4 * hd)

    y0l, y0r = t_first(base_emb, W0)
    z0l, z0r = sc(y0l, y0r)
    x1, y1l, y1r = t_mid_nores(z0l, z0r, W1)
    z1l, z1r = sc(y1l, y1r)
    x2, y2l, y2r = t_mid_res(z1l, z1r, x1, W2)
    z2l, z2r = sc(y2l, y2r)
    return t_final(z2l, z2r, x2, base_emb, x1, Wc, bc2)


# speedup vs baseline: 1.3082x; 1.3082x over previous
import functools

import jax
import jax.numpy as jnp
from jax import lax
from jax.experimental import pallas as pl
from jax.experimental.pallas import tpu as pltpu
from jax.experimental.pallas import tpu_sc as plsc

NSUB = 16
NCORE = 2
LANES = 16


def _splat(vec16, j):
    return lax.gather(
        vec16, jnp.full((LANES, 1), j, jnp.int32),
        lax.GatherDimensionNumbers(offset_dims=(), collapsed_slice_dims=(0,),
                                   start_index_map=(0,)),
        (1,), mode=lax.GatherScatterMode.PROMISE_IN_BOUNDS)


CH = 128
NBUF = 4


def _chunks_per_tile(e):
    return -(-e // (NSUB * CH * 2 * NBUF)) * 2 * NBUF


@functools.lru_cache(maxsize=None)
def _make_spmm(n, e, hd):
    assert n % NSUB == 0 and hd % LANES == 0
    cpt = _chunks_per_tile(e)
    nsteps = cpt // (2 * NBUF)
    rpa = -(-(n // NSUB) // 8) * 8
    rpl = n - (NSUB - 1) * rpa
    assert 0 < rpl <= rpa and rpl % 8 == 0
    nslice = hd // LANES

    mesh = plsc.VectorSubcoreMesh(core_axis_name="c", subcore_axis_name="s",
                                  num_cores=NCORE, num_subcores=NSUB)

    @functools.partial(
        pl.kernel,
        out_type=(jax.ShapeDtypeStruct((n, hd), jnp.float32),
                  jax.ShapeDtypeStruct((n, hd), jnp.float32)),
        mesh=mesh,
        scratch_types=[
            pltpu.VMEM_SHARED((n, hd), jnp.float32),
            [pltpu.VMEM((NBUF, CH), jnp.int32) for _ in range(3)],
            [pltpu.VMEM((NBUF, CH), jnp.int32) for _ in range(3)],
            [pltpu.VMEM((CH, hd), jnp.float32) for _ in range(NBUF)],
            [pltpu.SemaphoreType.DMA for _ in range(NBUF)],
            [pltpu.SemaphoreType.DMA for _ in range(NBUF)],
            pltpu.SemaphoreType.DMA,
            pltpu.SemaphoreType.DMA,
        ],
        compiler_params=pltpu.CompilerParams(use_tc_tiling_on_sc=False,
                                             needs_layout_passes=False),
    )
    def spmm(yl, yr, rows_h, cols_h, w_h, zero_h, zl, zr,
             acc, ib0, ib1, msgs, gsems, ssems, isem0, isem1):
        idx_h = (rows_h, cols_h, w_h)
        sub = lax.axis_index("s")
        core = lax.axis_index("c")
        cbase = sub * cpt

        @pl.when(sub < NSUB - 1)
        def _():
            pltpu.sync_copy(zero_h, acc.at[pl.ds(sub * rpa, rpa)])

        @pl.when(sub == NSUB - 1)
        def _():
            pltpu.sync_copy(zero_h.at[pl.ds(0, rpl)],
                            acc.at[pl.ds(sub * rpa, rpl)])
        plsc.subcore_barrier()

        def scale(msg, ib, b):
            def group(g, carry):
                w16 = plsc.bitcast(ib[2][b, pl.ds(g * LANES, LANES)],
                                   jnp.float32)
                for j in range(LANES):
                    k = g * LANES + j
                    s = _splat(w16, j)
                    for c in range(nslice):
                        sl = pl.ds(c * LANES, LANES)
                        msg[k, sl] = msg[k, sl] * s
                return carry
            lax.fori_loop(0, CH // LANES, group, 0)

        for half in range(NCORE):
            @pl.when(core == half)
            def _():
                ytab = (yl, yr)[half]

                def drain_scatter(b):
                    pltpu.make_async_copy(zero_h.at[pl.ds(0, CH)], msgs[b],
                                          ssems[b]).wait()

                def halfstep(ib, ib_other, isem_other, base_other):
                    descs = []
                    for b in range(NBUF):
                        drain_scatter(b)
                        descs.append(pltpu.async_copy(
                            ytab.at[ib[1].at[b]], msgs[b], gsems[b]))
                    prefs = [pltpu.async_copy(
                        idx_h[t].at[pl.ds(base_other, NBUF)],
                        ib_other[t], isem_other) for t in range(3)]
                    for b in range(NBUF):
                        descs[b].wait()
                        scale(msgs[b], ib, b)
                        pltpu.async_copy(msgs[b], acc.at[ib[0].at[b]],
                                         ssems[b], add=True)
                    for p in prefs:
                        p.wait()

                def step(j, carry):
                    base = cbase + 2 * NBUF * j
                    halfstep(ib0, ib1, isem1, base + NBUF)
                    halfstep(ib1, ib0, isem0, base + 2 * NBUF)
                    return carry

                for b in range(NBUF):
                    pltpu.async_copy(zero_h.at[pl.ds(0, CH)], msgs[b],
                                     ssems[b])
                for t in range(3):
                    pltpu.sync_copy(idx_h[t].at[pl.ds(cbase, NBUF)], ib0[t])
                lax.fori_loop(0, nsteps, step, 0)
                for b in range(NBUF):
                    drain_scatter(b)

        plsc.subcore_barrier()
        for half, zout in enumerate((zl, zr)):
            @pl.when((core == half) & (sub < NSUB - 1))
            def _():
                pltpu.sync_copy(acc.at[pl.ds(sub * rpa, rpa)],
                                zout.at[pl.ds(sub * rpa, rpa)])

            @pl.when((core == half) & (sub == NSUB - 1))
            def _():
                pltpu.sync_copy(acc.at[pl.ds(sub * rpa, rpl)],
                                zout.at[pl.ds(sub * rpa, rpl)])

    return spmm


def _dot(a, b):
    return jax.lax.dot_general(
        a, b, (((1,), (0,)), ((), ())),
        precision=jax.lax.Precision.HIGHEST,
        preferred_element_type=jnp.float32)


def _pack_half(y4, lo, hd):
    return jnp.concatenate([y4[:, i, lo:lo + hd] for i in range(4)], axis=1)


def _unpack_z(zl, zr, bm, hd):
    quarters = [
        jnp.concatenate([zl[:, i * hd:(i + 1) * hd],
                         zr[:, i * hd:(i + 1) * hd]], axis=1)
        for i in range(4)]
    return jnp.stack(quarters, axis=1).reshape(bm, 2 * hd)


def _pack_x(x, bm, d):
    xv = x.reshape(bm // 2, 2, d)
    return jnp.concatenate([xv[:, 0, :], xv[:, 1, :]], axis=1)


def _unpack_x(xp, bm, d):
    return jnp.stack([xp[:, :d], xp[:, d:]], axis=1).reshape(bm, d)


@functools.lru_cache(maxsize=None)
def _make_t_first(n, d, bm):
    hd = d // 2

    def body(x_ref, w_ref, yl_ref, yr_ref):
        y = _dot(x_ref[...], w_ref[...].T)
        y4 = y.reshape(bm // 4, 4, d)
        yl_ref[...] = _pack_half(y4, 0, hd)
        yr_ref[...] = _pack_half(y4, hd, hd)

    return pl.pallas_call(
        body,
        grid=(-(-n // bm),),
        in_specs=[pl.BlockSpec((bm, d), lambda i: (i, 0)),
                  pl.BlockSpec((d, d), lambda i: (0, 0))],
        out_specs=[pl.BlockSpec((bm // 4, 4 * hd), lambda i: (i, 0)),
                   pl.BlockSpec((bm // 4, 4 * hd), lambda i: (i, 0))],
        out_shape=[jax.ShapeDtypeStruct((n // 4, 4 * hd), jnp.float32),
                   jax.ShapeDtypeStruct((n // 4, 4 * hd), jnp.float32)],
        compiler_params=pltpu.CompilerParams(
            dimension_semantics=("parallel",)),
    )


@functools.lru_cache(maxsize=None)
def _make_t_mid(n, d, bm, residual):
    hd = d // 2

    def body(*refs):
        if residual:
            zl_ref, zr_ref, xp_ref, w_ref, x_ref, yl_ref, yr_ref = refs
        else:
            zl_ref, zr_ref, w_ref, x_ref, yl_ref, yr_ref = refs
        z = _unpack_z(zl_ref[...], zr_ref[...], bm, hd)
        x = jnp.maximum(z, 0.0)
        if residual:
            x = x + xp_ref[...]
        x_ref[...] = x
        y = _dot(x, w_ref[...].T)
        y4 = y.reshape(bm // 4, 4, d)
        yl_ref[...] = _pack_half(y4, 0, hd)
        yr_ref[...] = _pack_half(y4, hd, hd)

    in_specs = [pl.BlockSpec((bm // 4, 4 * hd), lambda i: (i, 0)),
                pl.BlockSpec((bm // 4, 4 * hd), lambda i: (i, 0))]
    if residual:
        in_specs.append(pl.BlockSpec((bm, d), lambda i: (i, 0)))
    in_specs.append(pl.BlockSpec((d, d), lambda i: (0, 0)))

    return pl.pallas_call(
        body,
        grid=(-(-n // bm),),
        in_specs=in_specs,
        out_specs=[pl.BlockSpec((bm, d), lambda i: (i, 0)),
                   pl.BlockSpec((bm // 4, 4 * hd), lambda i: (i, 0)),
                   pl.BlockSpec((bm // 4, 4 * hd), lambda i: (i, 0))],
        out_shape=[jax.ShapeDtypeStruct((n, d), jnp.float32),
                   jax.ShapeDtypeStruct((n // 4, 4 * hd), jnp.float32),
                   jax.ShapeDtypeStruct((n // 4, 4 * hd), jnp.float32)],
        compiler_params=pltpu.CompilerParams(
            dimension_semantics=("parallel",)),
    )


@functools.lru_cache(maxsize=None)
def _make_t_final(n, d, bm):
    hd = d // 2

    def body(zl_ref, zr_ref, x2_ref, x0_ref, x1_ref, wc_ref, bc_ref, o_ref):
        z = _unpack_z(zl_ref[...], zr_ref[...], bm, hd)
        x3 = jnp.maximum(z, 0.0) + x2_ref[...]
        comb = jnp.concatenate(
            [x0_ref[...], x1_ref[...], x2_ref[...], x3], axis=1)
        o_ref[...] = _dot(comb, wc_ref[...].T) + bc_ref[...]

    return pl.pallas_call(
        body,
        grid=(-(-n // bm),),
        in_specs=[pl.BlockSpec((bm // 4, 4 * hd), lambda i: (i, 0)),
                  pl.BlockSpec((bm // 4, 4 * hd), lambda i: (i, 0)),
                  pl.BlockSpec((bm, d), lambda i: (i, 0)),
                  pl.BlockSpec((bm, d), lambda i: (i, 0)),
                  pl.BlockSpec((bm, d), lambda i: (i, 0)),
                  pl.BlockSpec((d, 4 * d), lambda i: (0, 0)),
                  pl.BlockSpec((1, d), lambda i: (0, 0))],
        out_specs=pl.BlockSpec((bm, d), lambda i: (i, 0)),
        out_shape=jax.ShapeDtypeStruct((n, d), jnp.float32),
        compiler_params=pltpu.CompilerParams(
            dimension_semantics=("parallel",)),
    )


def kernel(base_emb, edge_index, edge_weight, W0, W1, W2, Wc, bc):
    n, d = base_emb.shape
    e = edge_weight.shape[0]
    hd = d // 2
    bm = 1024

    cpt = _chunks_per_tile(e)
    e2 = NSUB * cpt * CH
    pad = e2 - e + 2 * NBUF * CH
    rows = jnp.pad(edge_index[0], (0, pad)).reshape(-1, CH)
    cols = jnp.pad(edge_index[1], (0, pad)).reshape(-1, CH)
    wbits = jnp.pad(lax.bitcast_convert_type(edge_weight, jnp.int32),
                    (0, pad)).reshape(-1, CH)
    zero = jnp.zeros((-(-(n // NSUB) // 8) * 8, hd), jnp.float32)
    bc2 = bc.reshape(1, d)

    spmm = _make_spmm(n, e, hd)
    t_first = _make_t_first(n, d, bm)
    t_mid_nores = _make_t_mid(n, d, bm, False)
    t_mid_res = _make_t_mid(n, d, bm, True)
    t_final = _make_t_final(n, d, bm)

    def sc(ylp, yrp):
        zl, zr = spmm(ylp.reshape(n, hd), yrp.reshape(n, hd),
                      rows, cols, wbits, zero)
        return zl.reshape(n // 4, 4 * hd), zr.reshape(n // 4, 4 * hd)

    y0l, y0r = t_first(base_emb, W0)
    z0l, z0r = sc(y0l, y0r)
    x1, y1l, y1r = t_mid_nores(z0l, z0r, W1)
    z1l, z1r = sc(y1l, y1r)
    x2, y2l, y2r = t_mid_res(z1l, z1r, x1, W2)
    z2l, z2r = sc(y2l, y2r)
    return t_final(z2l, z2r, x2, base_emb, x1, Wc, bc2)
